# baseline (speedup 1.0000x reference)
import jax
import jax.numpy as jnp
from jax import lax
from jax.experimental import pallas as pl
from jax.experimental.pallas import tpu as pltpu
from jax.experimental.pallas import tpu_sc as plsc

N_NODES = 10000
N_EDGES = 320000
D = 128
DH = D // 2
R = 8

NC = 2
NS = 16

CHUNK = 80
RB = 10
NBLK = N_EDGES // (CHUNK * RB * NS)
N_PAD = 10240
NODES_PER_TILE = N_PAD // NS


def _mm_body(feat_ref, w_ref, out_ref):
    out_ref[0, 0] = jnp.dot(feat_ref[...], w_ref[0, 0],
                            preferred_element_type=jnp.float32)


def _typed_transform(feat, W):
    BN = 1000
    NB = N_NODES // BN
    W4 = W.reshape(R, D, NC, DH).transpose(2, 0, 1, 3)
    return pl.pallas_call(
        _mm_body,
        grid=(NB, R, NC),
        in_specs=[
            pl.BlockSpec((BN, D), lambda n, r, h: (n, 0)),
            pl.BlockSpec((1, 1, D, DH), lambda n, r, h: (h, r, 0, 0)),
        ],
        out_specs=pl.BlockSpec((1, 1, BN, DH), lambda n, r, h: (h, r, n, 0)),
        out_shape=jax.ShapeDtypeStruct((NC, R, N_NODES, DH), jnp.float32),
    )(feat, W4)


def _sc_body(table, src4, dst4, et4, out, src_v, dst_v, et_v, gidx_v,
             rows_a, rows_b, shared_acc, sem_a, sem_b):
    c = lax.axis_index("c")
    s = lax.axis_index("s")
    base = s * NODES_PER_TILE
    bufs = (rows_a, rows_b)
    sems = (sem_a, sem_b)
    my_table = table.at[c]

    def _zero_row(j, _):
        for k in range(DH // 16):
            rows_a[j, pl.ds(k * 16, 16)] = jnp.zeros((16,), jnp.float32)
        return 0
    lax.fori_loop(0, CHUNK, _zero_row, 0)
    for i in range(NODES_PER_TILE // CHUNK):
        pltpu.sync_copy(rows_a, shared_acc.at[pl.ds(base + i * CHUNK, CHUNK)])

    plsc.subcore_barrier()

    def _block(b, _):
        pltpu.sync_copy(src4.at[s, b], src_v)
        pltpu.sync_copy(dst4.at[s, b], dst_v)
        pltpu.sync_copy(et4.at[s, b], et_v)

        def _gidx_row(j, _):
            for k in range(CHUNK // 16):
                sl = pl.ds(k * 16, 16)
                gidx_v[j, sl] = et_v[j, sl] * N_NODES + src_v[j, sl]
            return 0
        lax.fori_loop(0, RB, _gidx_row, 0)

        copies = [None] * RB
        copies[0] = pltpu.async_copy(my_table.at[gidx_v.at[0]],
                                     bufs[0], sems[0])
        for r in range(1, RB):
            copies[r] = pltpu.async_copy(my_table.at[gidx_v.at[r]],
                                         bufs[r % 2], sems[r % 2])
            copies[r - 1].wait()
            pltpu.sync_copy(bufs[(r - 1) % 2],
                            shared_acc.at[dst_v.at[r - 1]], add=True)
        copies[RB - 1].wait()
        pltpu.sync_copy(bufs[(RB - 1) % 2],
                        shared_acc.at[dst_v.at[RB - 1]], add=True)
        return 0
    lax.fori_loop(0, NBLK, _block, 0)

    plsc.subcore_barrier()

    pltpu.sync_copy(shared_acc.at[pl.ds(base, NODES_PER_TILE)],
                    out.at[c, pl.ds(base, NODES_PER_TILE)])


def _sc_scatter(table3, src4, dst4, et4):
    mesh = plsc.VectorSubcoreMesh(core_axis_name="c", subcore_axis_name="s",
                                  num_cores=NC)
    return pl.kernel(
        _sc_body,
        out_type=jax.ShapeDtypeStruct((NC, N_PAD, DH), jnp.float32),
        mesh=mesh,
        compiler_params=pltpu.CompilerParams(use_tc_tiling_on_sc=False),
        scratch_types=[
            pltpu.VMEM((RB, CHUNK), jnp.int32),
            pltpu.VMEM((RB, CHUNK), jnp.int32),
            pltpu.VMEM((RB, CHUNK), jnp.int32),
            pltpu.VMEM((RB, CHUNK), jnp.int32),
            pltpu.VMEM((CHUNK, DH), jnp.float32),
            pltpu.VMEM((CHUNK, DH), jnp.float32),
            pltpu.VMEM_SHARED((N_PAD, DH), jnp.float32),
            pltpu.SemaphoreType.DMA,
            pltpu.SemaphoreType.DMA,
        ],
    )(table3, src4, dst4, et4)


def _add_body(f_ref, p0_ref, p1_ref, o_ref):
    o_ref[...] = f_ref[...] + jnp.concatenate([p0_ref[...], p1_ref[...]],
                                              axis=1)


def _final_add(feat, p0, p1):
    BN = 1000
    NB = N_NODES // BN
    spec = pl.BlockSpec((BN, D), lambda n: (n, 0))
    hspec = pl.BlockSpec((BN, DH), lambda n: (n, 0))
    return pl.pallas_call(
        _add_body,
        grid=(NB,),
        in_specs=[spec, hspec, hspec],
        out_specs=spec,
        out_shape=jax.ShapeDtypeStruct((N_NODES, D), jnp.float32),
    )(feat, p0, p1)


@jax.jit
def kernel(feat, edge_index, etypes, W):
    table3 = _typed_transform(feat, W).reshape(NC, R * N_NODES, DH)
    src4 = edge_index[0].reshape(NS, NBLK, RB, CHUNK)
    dst4 = edge_index[1].reshape(NS, NBLK, RB, CHUNK)
    et4 = etypes.reshape(NS, NBLK, RB, CHUNK).astype(jnp.int32)
    partials = _sc_scatter(table3, src4, dst4, et4)
    return _final_add(feat, partials[0, :N_NODES], partials[1, :N_NODES])


# speedup vs baseline: 1.2541x; 1.2541x over previous
import jax
import jax.numpy as jnp
from jax import lax
from jax.experimental import pallas as pl
from jax.experimental.pallas import tpu as pltpu
from jax.experimental.pallas import tpu_sc as plsc

N_NODES = 10000
N_EDGES = 320000
D = 128
R = 8

NC = 1
NS = 16
NW = NC * NS

CHUNK = 80
RB = 10
NBLK = N_EDGES // (CHUNK * RB * NW)
N_PAD = 10240
NODES_PER_TILE = N_PAD // NS
NBUF = 3


def _mm_body(feat_ref, w_ref, out_ref):
    out_ref[0] = jnp.dot(feat_ref[...], w_ref[0],
                         preferred_element_type=jnp.float32)


def _typed_transform(feat, W):
    BN = 1000
    NB = N_NODES // BN
    return pl.pallas_call(
        _mm_body,
        grid=(NB, R),
        in_specs=[
            pl.BlockSpec((BN, D), lambda n, r: (n, 0)),
            pl.BlockSpec((1, D, D), lambda n, r: (r, 0, 0)),
        ],
        out_specs=pl.BlockSpec((1, BN, D), lambda n, r: (r, n, 0)),
        out_shape=jax.ShapeDtypeStruct((R, N_NODES, D), jnp.float32),
    )(feat, W)


def _sc_body(table, feat, src4, dst4, et4, out, src_v, dst_v, et_v, gidx_v,
             rows_a, rows_b, rows_c, shared_acc,
             gsem_a, gsem_b, gsem_c, ssem_a, ssem_b, ssem_c):
    s = lax.axis_index("s")
    base = s * NODES_PER_TILE
    bufs = (rows_a, rows_b, rows_c)
    gsems = (gsem_a, gsem_b, gsem_c)
    ssems = (ssem_a, ssem_b, ssem_c)

    @pl.when(s < NS - 1)
    def _():
        pltpu.sync_copy(feat.at[pl.ds(base, NODES_PER_TILE)],
                        shared_acc.at[pl.ds(base, NODES_PER_TILE)])

    @pl.when(s == NS - 1)
    def _():
        pltpu.sync_copy(feat.at[pl.ds(base, N_NODES - (NS - 1) * NODES_PER_TILE)],
                        shared_acc.at[pl.ds(base, N_NODES - (NS - 1) * NODES_PER_TILE)])

    plsc.subcore_barrier()

    def _block(b, _):
        pltpu.sync_copy(src4.at[s, b], src_v)
        pltpu.sync_copy(dst4.at[s, b], dst_v)
        pltpu.sync_copy(et4.at[s, b], et_v)

        def _gidx_row(j, _):
            for k in range(CHUNK // 16):
                sl = pl.ds(k * 16, 16)
                gidx_v[j, sl] = et_v[j, sl] * N_NODES + src_v[j, sl]
            return 0
        lax.fori_loop(0, RB, _gidx_row, 0)

        g = [None] * RB
        sc = [None] * RB
        for r in range(RB):
            if r >= NBUF:
                sc[r - NBUF].wait()
            g[r] = pltpu.async_copy(table.at[gidx_v.at[r]],
                                    bufs[r % NBUF], gsems[r % NBUF])
            if r >= 1:
                g[r - 1].wait()
                sc[r - 1] = pltpu.async_copy(
                    bufs[(r - 1) % NBUF],
                    shared_acc.at[dst_v.at[r - 1]],
                    ssems[(r - 1) % NBUF], add=True)
        g[RB - 1].wait()
        sc[RB - 1] = pltpu.async_copy(
            bufs[(RB - 1) % NBUF],
            shared_acc.at[dst_v.at[RB - 1]],
            ssems[(RB - 1) % NBUF], add=True)
        for r in range(RB - NBUF, RB):
            sc[r].wait()
        return 0
    lax.fori_loop(0, NBLK, _block, 0)

    plsc.subcore_barrier()

    @pl.when(s < NS - 1)
    def _():
        pltpu.sync_copy(shared_acc.at[pl.ds(base, NODES_PER_TILE)],
                        out.at[pl.ds(base, NODES_PER_TILE)])

    @pl.when(s == NS - 1)
    def _():
        pltpu.sync_copy(shared_acc.at[pl.ds(base, N_NODES - (NS - 1) * NODES_PER_TILE)],
                        out.at[pl.ds(base, N_NODES - (NS - 1) * NODES_PER_TILE)])


def _sc_scatter(table2d, feat, src4, dst4, et4):
    mesh = plsc.VectorSubcoreMesh(core_axis_name="c", subcore_axis_name="s",
                                  num_cores=NC)
    return pl.kernel(
        _sc_body,
        out_type=jax.ShapeDtypeStruct((N_NODES, D), jnp.float32),
        mesh=mesh,
        scratch_types=[
            pltpu.VMEM((RB, CHUNK), jnp.int32),
            pltpu.VMEM((RB, CHUNK), jnp.int32),
            pltpu.VMEM((RB, CHUNK), jnp.int32),
            pltpu.VMEM((RB, CHUNK), jnp.int32),
            pltpu.VMEM((CHUNK, D), jnp.float32),
            pltpu.VMEM((CHUNK, D), jnp.float32),
            pltpu.VMEM((CHUNK, D), jnp.float32),
            pltpu.VMEM_SHARED((N_PAD, D), jnp.float32),
            pltpu.SemaphoreType.DMA,
            pltpu.SemaphoreType.DMA,
            pltpu.SemaphoreType.DMA,
            pltpu.SemaphoreType.DMA,
            pltpu.SemaphoreType.DMA,
            pltpu.SemaphoreType.DMA,
        ],
    )(table2d, feat, src4, dst4, et4)


@jax.jit
def kernel(feat, edge_index, etypes, W):
    table = _typed_transform(feat, W).reshape(R * N_NODES, D)
    src4 = edge_index[0].reshape(NW, NBLK, RB, CHUNK)
    dst4 = edge_index[1].reshape(NW, NBLK, RB, CHUNK)
    et4 = etypes.reshape(NW, NBLK, RB, CHUNK).astype(jnp.int32)
    return _sc_scatter(table, feat, src4, dst4, et4)


# speedup vs baseline: 1.5508x; 1.2365x over previous
import jax
import jax.numpy as jnp
from jax import lax
from jax.experimental import pallas as pl
from jax.experimental.pallas import tpu as pltpu
from jax.experimental.pallas import tpu_sc as plsc

N_NODES = 10000
N_EDGES = 320000
D = 128
R = 8

NC = 1
NS = 16
NW = NC * NS

CHUNK = 80
RB = 50
NBLK = N_EDGES // (CHUNK * RB * NW)
N_PAD = 10240
NODES_PER_TILE = N_PAD // NS
NODES_LAST = N_NODES - (NS - 1) * NODES_PER_TILE
NBUF = 3


def _mm_body(feat_ref, w_ref, out_ref):
    out_ref[0] = jnp.dot(feat_ref[...], w_ref[0],
                         preferred_element_type=jnp.float32)


def _typed_transform(feat, W):
    BN = 1000
    NB = N_NODES // BN
    return pl.pallas_call(
        _mm_body,
        grid=(NB, R),
        in_specs=[
            pl.BlockSpec((BN, D), lambda n, r: (n, 0)),
            pl.BlockSpec((1, D, D), lambda n, r: (r, 0, 0)),
        ],
        out_specs=pl.BlockSpec((1, BN, D), lambda n, r: (r, n, 0)),
        out_shape=jax.ShapeDtypeStruct((R, N_NODES, D), jnp.float32),
    )(feat, W)


def _sc_body(table, feat, gidx4, dst4, out, gidx_v, dst_v,
             rows_a, rows_b, rows_c, shared_acc,
             gsem_a, gsem_b, gsem_c, ssem_a, ssem_b, ssem_c):
    s = lax.axis_index("s")
    base = s * NODES_PER_TILE
    bufs = (rows_a, rows_b, rows_c)
    gsems = (gsem_a, gsem_b, gsem_c)
    ssems = (ssem_a, ssem_b, ssem_c)

    @pl.when(s < NS - 1)
    def _():
        pltpu.sync_copy(feat.at[pl.ds(base, NODES_PER_TILE)],
                        shared_acc.at[pl.ds(base, NODES_PER_TILE)])

    @pl.when(s == NS - 1)
    def _():
        pltpu.sync_copy(feat.at[pl.ds(base, NODES_LAST)],
                        shared_acc.at[pl.ds(base, NODES_LAST)])

    plsc.subcore_barrier()

    def _block(b, _):
        pltpu.sync_copy(gidx4.at[s, b], gidx_v)
        pltpu.sync_copy(dst4.at[s, b], dst_v)

        g = [None] * RB
        sc = [None] * RB
        for r in range(RB):
            if r >= NBUF:
                sc[r - NBUF].wait()
            g[r] = pltpu.async_copy(table.at[gidx_v.at[r]],
                                    bufs[r % NBUF], gsems[r % NBUF])
            if r >= 1:
                g[r - 1].wait()
                sc[r - 1] = pltpu.async_copy(
                    bufs[(r - 1) % NBUF],
                    shared_acc.at[dst_v.at[r - 1]],
                    ssems[(r - 1) % NBUF], add=True)
        g[RB - 1].wait()
        sc[RB - 1] = pltpu.async_copy(
            bufs[(RB - 1) % NBUF],
            shared_acc.at[dst_v.at[RB - 1]],
            ssems[(RB - 1) % NBUF], add=True)
        for r in range(RB - NBUF, RB):
            sc[r].wait()
        return 0
    lax.fori_loop(0, NBLK, _block, 0)

    plsc.subcore_barrier()

    @pl.when(s < NS - 1)
    def _():
        pltpu.sync_copy(shared_acc.at[pl.ds(base, NODES_PER_TILE)],
                        out.at[pl.ds(base, NODES_PER_TILE)])

    @pl.when(s == NS - 1)
    def _():
        pltpu.sync_copy(shared_acc.at[pl.ds(base, NODES_LAST)],
                        out.at[pl.ds(base, NODES_LAST)])


def _sc_scatter(table2d, feat, gidx4, dst4):
    mesh = plsc.VectorSubcoreMesh(core_axis_name="c", subcore_axis_name="s",
                                  num_cores=NC)
    return pl.kernel(
        _sc_body,
        out_type=jax.ShapeDtypeStruct((N_NODES, D), jnp.float32),
        mesh=mesh,
        scratch_types=[
            pltpu.VMEM((RB, CHUNK), jnp.int32),
            pltpu.VMEM((RB, CHUNK), jnp.int32),
            pltpu.VMEM((CHUNK, D), jnp.float32),
            pltpu.VMEM((CHUNK, D), jnp.float32),
            pltpu.VMEM((CHUNK, D), jnp.float32),
            pltpu.VMEM_SHARED((N_PAD, D), jnp.float32),
            pltpu.SemaphoreType.DMA,
            pltpu.SemaphoreType.DMA,
            pltpu.SemaphoreType.DMA,
            pltpu.SemaphoreType.DMA,
            pltpu.SemaphoreType.DMA,
            pltpu.SemaphoreType.DMA,
        ],
    )(table2d, feat, gidx4, dst4)


@jax.jit
def kernel(feat, edge_index, etypes, W):
    table = _typed_transform(feat, W).reshape(R * N_NODES, D)
    gidx4 = (etypes.astype(jnp.int32) * N_NODES
             + edge_index[0]).reshape(NW, NBLK, RB, CHUNK)
    dst4 = edge_index[1].reshape(NW, NBLK, RB, CHUNK)
    return _sc_scatter(table, feat, gidx4, dst4)
